# baseline (speedup 1.0000x reference)
import functools

import jax
import jax.numpy as jnp
from jax import lax
from jax.experimental import pallas as pl
from jax.experimental.pallas import tpu as pltpu
from jax.experimental.pallas import tpu_sc as plsc

N = 8388608
NC = 2
NS = 16
L = 16
NW = NC * NS
PER_W = N // NW
CHUNK = 16384
NCHUNK = PER_W // CHUNK
UNROLL = 8
NBINS = 10

_mesh = plsc.VectorSubcoreMesh(core_axis_name="c", subcore_axis_name="s")


@functools.partial(
    pl.kernel,
    out_type=(
        jax.ShapeDtypeStruct((NBINS * NW * L,), jnp.float32),
        jax.ShapeDtypeStruct((NBINS * NW * L,), jnp.float32),
    ),
    mesh=_mesh,
    compiler_params=pltpu.CompilerParams(needs_layout_passes=False),
    scratch_types=(
        pltpu.VMEM((2, CHUNK), jnp.float32),
        pltpu.VMEM((2, CHUNK), jnp.float32),
        pltpu.VMEM((NBINS * L,), jnp.float32),
        pltpu.VMEM((NBINS * L,), jnp.float32),
        pltpu.SemaphoreType.DMA,
        pltpu.SemaphoreType.DMA,
        pltpu.SemaphoreType.DMA,
    ),
)
def _histo(pred_hbm, tgt_hbm, cnt_out, sum_out, pbuf, tbuf, cacc, sacc,
           sem0, sem1, sem_out):
    wid = lax.axis_index("c") * NS + lax.axis_index("s")
    base = wid * PER_W

    zeros = jnp.zeros((L,), jnp.float32)
    for j in range(NBINS):
        cacc[pl.ds(j * L, L)] = zeros
        sacc[pl.ds(j * L, L)] = zeros

    sems = (sem0, sem1)

    def start(g):
        slot = g % 2
        lo = base + g * CHUNK
        return (
            pltpu.async_copy(pred_hbm.at[pl.ds(lo, CHUNK)], pbuf.at[slot],
                             sems[slot]),
            pltpu.async_copy(tgt_hbm.at[pl.ds(lo, CHUNK)], tbuf.at[slot],
                             sems[slot]),
        )

    lanes = lax.iota(jnp.int32, L)
    ones = jnp.ones((L,), jnp.float32)
    half = jnp.float32(0.5)
    threehalf = jnp.float32(1.5)
    ten = jnp.float32(10.0)
    tiny = jnp.float32(1e-30)
    magic = jnp.int32(0x5F3759DF)

    pending = start(0)
    for g in range(NCHUNK):
        nxt = start(g + 1) if g + 1 < NCHUNK else None
        for h in pending:
            h.wait()
        slot = g % 2

        def body(k, carry):
            off = k * (UNROLL * L)
            for u in range(UNROLL):
                p = pbuf[slot, pl.ds(off + u * L, L)]
                t = tbuf[slot, pl.ds(off + u * L, L)]
                d = jnp.abs(p - t)
                dc = jnp.maximum(d, tiny)
                y = lax.bitcast_convert_type(
                    magic - (lax.bitcast_convert_type(dc, jnp.int32) >> 1),
                    jnp.float32)
                xh = half * dc
                y = y * (threehalf - xh * y * y)
                y = y * (threehalf - xh * y * y)
                gg = d * y
                v = d * gg
                b = jnp.minimum((gg * ten).astype(jnp.int32), 9)
                idx = (b << 4) + lanes
                plsc.addupdate_scatter(cacc, [idx], ones)
                plsc.addupdate_scatter(sacc, [idx], v)
            return carry

        lax.fori_loop(0, CHUNK // (UNROLL * L), body, jnp.int32(0))
        pending = nxt

    hs = []
    for j in range(NBINS):
        src = pl.ds(j * L, L)
        dst = pl.ds(j * NW * L + wid * L, L)
        hs.append(pltpu.async_copy(cacc.at[src], cnt_out.at[dst], sem_out))
        hs.append(pltpu.async_copy(sacc.at[src], sum_out.at[dst], sem_out))
    for h in hs:
        h.wait()


def _finalize_body(cnt_ref, sum_ref, o_ref):
    c = cnt_ref[...]
    s = sum_ref[...]
    cb = jnp.sum(c, axis=1, keepdims=True)
    sb = jnp.sum(s, axis=1, keepdims=True)
    nz = cb > 0.0
    n = jnp.sum(nz.astype(jnp.float32))
    contrib = jnp.where(nz, sb / jnp.maximum(cb, 1.0), 0.0)
    o_ref[0, 0] = jnp.sum(contrib) / jnp.maximum(n, 1.0)


def kernel(pred, target):
    cnt, sm = _histo(pred, target)
    cnt = cnt.reshape(NBINS, NW * L)
    sm = sm.reshape(NBINS, NW * L)
    out = pl.pallas_call(
        _finalize_body,
        out_shape=jax.ShapeDtypeStruct((1, 1), jnp.float32),
        out_specs=pl.BlockSpec(memory_space=pltpu.SMEM),
    )(cnt, sm)
    return out[0, 0]


# speedup vs baseline: 4.2132x; 4.2132x over previous
import functools

import numpy as np

import jax
import jax.numpy as jnp
from jax import lax
from jax.experimental import pallas as pl
from jax.experimental.pallas import tpu as pltpu
from jax.experimental.pallas import tpu_sc as plsc

N = 8388608
NC = 2
NS = 16
L = 16
NW = NC * NS
CHUNK = 16384
PER_W = N // NW
NCHUNK = PER_W // CHUNK
UNROLL = 16
NBINS = 10

_mesh = plsc.VectorSubcoreMesh(core_axis_name="c", subcore_axis_name="s")


@functools.partial(
    pl.kernel,
    out_type=(
        jax.ShapeDtypeStruct((NBINS * NW * L,), jnp.float32),
        jax.ShapeDtypeStruct((NBINS * NW * L,), jnp.float32),
    ),
    mesh=_mesh,
    compiler_params=pltpu.CompilerParams(needs_layout_passes=False),
    scratch_types=(
        pltpu.VMEM((2, CHUNK), jnp.float32),
        pltpu.VMEM((2, CHUNK), jnp.float32),
        pltpu.VMEM((4 * (NBINS + 1) * L,), jnp.float32),
        pltpu.VMEM((4 * (NBINS + 1) * L,), jnp.float32),
        pltpu.SemaphoreType.DMA,
        pltpu.SemaphoreType.DMA,
        pltpu.SemaphoreType.DMA,
    ),
)
def _histo(pred_hbm, tgt_hbm, cnt_out, sum_out, pbuf, tbuf, cacc, sacc,
           sem0, sem1, sem_out):
    wid = lax.axis_index("c") * NS + lax.axis_index("s")
    base = wid * PER_W

    zeros = jnp.zeros((L,), jnp.float32)
    for j in range(4 * (NBINS + 1)):
        cacc[pl.ds(j * L, L)] = zeros
        sacc[pl.ds(j * L, L)] = zeros

    sems = (sem0, sem1)

    def start(g):
        slot = g % 2
        lo = base + g * CHUNK
        return (
            pltpu.async_copy(pred_hbm.at[pl.ds(lo, CHUNK)], pbuf.at[slot],
                             sems[slot]),
            pltpu.async_copy(tgt_hbm.at[pl.ds(lo, CHUNK)], tbuf.at[slot],
                             sems[slot]),
        )

    lanes = lax.iota(jnp.int32, L)
    STR = (NBINS + 1) * L
    lanes4 = [lanes + jnp.int32(c * STR) for c in range(4)]
    ones = jnp.ones((L,), jnp.float32)
    magic = jnp.int32(0x5F1FFFF9)
    c1 = jnp.float32(10.0 * 0.703952253 * 2.38924456)
    c2 = jnp.float32(10.0 * 0.703952253)
    U = UNROLL

    pending = start(0)
    for g in range(NCHUNK):
        nxt = start(g + 1) if g + 1 < NCHUNK else None
        for h in pending:
            h.wait()
        slot = g % 2

        def body(k, carry):
            off = k * (U * L)
            p = [pbuf[slot, pl.ds(off + u * L, L)] for u in range(U)]
            t = [tbuf[slot, pl.ds(off + u * L, L)] for u in range(U)]
            d = [jnp.abs(p[u] - t[u]) for u in range(U)]
            y = [lax.bitcast_convert_type(
                magic - (lax.bitcast_convert_type(d[u], jnp.int32) >> 1),
                jnp.float32) for u in range(U)]
            z = [d[u] * y[u] for u in range(U)]
            zy = [z[u] * y[u] for u in range(U)]
            w10 = [c1 - c2 * zy[u] for u in range(U)]
            bf = [z[u] * w10[u] for u in range(U)]
            v = [d[u] * bf[u] for u in range(U)]
            b = [bf[u].astype(jnp.int32) for u in range(U)]
            idx = [(b[u] << 4) + lanes4[u % 4] for u in range(U)]
            for u in range(U):
                plsc.addupdate_scatter(cacc, [idx[u]], ones)
                plsc.addupdate_scatter(sacc, [idx[u]], v[u])
            return carry

        lax.fori_loop(0, CHUNK // (U * L), body, jnp.int32(0))
        pending = nxt

    for j in range(NBINS):
        csum = cacc[pl.ds(j * L, L)]
        ssum = sacc[pl.ds(j * L, L)]
        for c in range(4):
            if c > 0:
                csum = csum + cacc[pl.ds(c * STR + j * L, L)]
                ssum = ssum + sacc[pl.ds(c * STR + j * L, L)]
            if j == 9:
                csum = csum + cacc[pl.ds(c * STR + 10 * L, L)]
                ssum = ssum + sacc[pl.ds(c * STR + 10 * L, L)]
        cacc[pl.ds(j * L, L)] = csum
        sacc[pl.ds(j * L, L)] = ssum

    hs = []
    for j in range(NBINS):
        src = pl.ds(j * L, L)
        dst = pl.ds(j * NW * L + wid * L, L)
        hs.append(pltpu.async_copy(cacc.at[src], cnt_out.at[dst], sem_out))
        hs.append(pltpu.async_copy(sacc.at[src], sum_out.at[dst], sem_out))
    for h in hs:
        h.wait()


def _finalize_body(cnt_ref, sum_ref, o_ref):
    c = cnt_ref[...]
    s = sum_ref[...]
    cb = jnp.sum(c, axis=1, keepdims=True)
    sb = jnp.float32(0.1) * jnp.sum(s, axis=1, keepdims=True)
    nz = cb > 0.0
    n = jnp.sum(nz.astype(jnp.float32))
    contrib = jnp.where(nz, sb / jnp.maximum(cb, 1.0), 0.0)
    o_ref[0, 0] = jnp.sum(contrib) / jnp.maximum(n, 1.0)


def kernel(pred, target):
    cnt, sm = _histo(pred, target)
    cnt = cnt.reshape(NBINS, NW * L)
    sm = sm.reshape(NBINS, NW * L)
    out = pl.pallas_call(
        _finalize_body,
        out_shape=jax.ShapeDtypeStruct((1, 1), jnp.float32),
        out_specs=pl.BlockSpec(memory_space=pltpu.SMEM),
    )(cnt, sm)
    return out[0, 0]
